# baseline (speedup 1.0000x reference)
import functools

import jax
import jax.numpy as jnp
from jax import lax
from jax.experimental import pallas as pl
from jax.experimental.pallas import tpu as pltpu
from jax.experimental.pallas import tpu_sc as plsc

_CH = 256


def _build(n, h, b):
    max_len = 256 * b
    mesh = plsc.VectorSubcoreMesh(core_axis_name="c", subcore_axis_name="s")
    nc = mesh.num_cores
    half = max_len // nc
    n_chunks = half // _CH

    @functools.partial(
        pl.kernel,
        out_type=jax.ShapeDtypeStruct((b, max_len, h), jnp.float32),
        mesh=mesh,
        scratch_types=[
            pltpu.VMEM((b,), jnp.int32),
            pltpu.VMEM((_CH, h), jnp.float32),
            pltpu.SemaphoreType.DMA,
            pltpu.SemaphoreType.DMA,
        ],
    )
    def run(words_hbm, dl_hbm, zpad_hbm, out_hbm, dl_v, zbuf, csem, zsem):
        s = lax.axis_index("s")
        c = lax.axis_index("c")

        pltpu.sync_copy(dl_hbm, dl_v)
        dl = dl_v[...]
        off_b = jnp.int32(0)
        len_b = jnp.int32(0)
        for i in range(b):
            dli = dl[i]
            off_b = off_b + jnp.where(i < s, dli, 0)
            len_b = len_b + jnp.where(i == s, dli, 0)
        off_b = pl.multiple_of(off_b, 8)

        p0 = c * half
        nvalid = jnp.clip(len_b - p0, 0, half)
        ncopy = nvalid // _CH

        @pl.when(ncopy < n_chunks)
        def _():
            pltpu.sync_copy(zpad_hbm, zbuf)

        for i in range(n_chunks):
            r0 = p0 + i * _CH
            src = words_hbm.at[pl.ds(off_b + r0, _CH), :]
            dst = out_hbm.at[s, pl.ds(r0, _CH), :]

            @pl.when(i < ncopy)
            def _(src=src, dst=dst):
                pltpu.async_copy(src, dst, csem)

            @pl.when(i >= ncopy)
            def _(dst=dst):
                pltpu.async_copy(zbuf, dst, zsem)

        for i in range(n_chunks):
            r0 = p0 + i * _CH
            src = words_hbm.at[pl.ds(off_b + r0, _CH), :]
            dst = out_hbm.at[s, pl.ds(r0, _CH), :]

            @pl.when(i < ncopy)
            def _(src=src, dst=dst):
                pltpu.make_async_copy(src, dst, csem).wait()

            @pl.when(i >= ncopy)
            def _(dst=dst):
                pltpu.make_async_copy(zbuf, dst, zsem).wait()

    return run


def kernel(words_out, doc_lens):
    n, h = words_out.shape
    b = doc_lens.shape[0]
    zpad = jnp.zeros((_CH, h), jnp.float32)
    run = _build(n, h, b)
    return run(words_out, jnp.asarray(doc_lens, jnp.int32), zpad)


# speedup vs baseline: 13.8087x; 13.8087x over previous
import functools

import jax
import jax.numpy as jnp
from jax import lax
from jax.experimental import pallas as pl
from jax.experimental.pallas import tpu as pltpu
from jax.experimental.pallas import tpu_sc as plsc

_CH = 256
_NB = 3
_ZR = 128


def _build(n, h, b):
    max_len = 256 * b
    mesh = plsc.VectorSubcoreMesh(core_axis_name="c", subcore_axis_name="s")
    nc = mesh.num_cores
    half = max_len // nc
    n_chunks = half // _CH

    @functools.partial(
        pl.kernel,
        out_type=jax.ShapeDtypeStruct((b, max_len, h), jnp.float32),
        mesh=mesh,
        scratch_types=[
            pltpu.VMEM((b,), jnp.int32),
            pltpu.VMEM((_ZR, h), jnp.float32),
            [pltpu.VMEM((_CH, h), jnp.float32) for _ in range(_NB)],
            [pltpu.SemaphoreType.DMA for _ in range(_NB)],
            [pltpu.SemaphoreType.DMA for _ in range(_NB)],
            pltpu.SemaphoreType.DMA,
        ],
    )
    def run(words_hbm, dl_hbm, zpad_hbm, out_hbm,
            dl_v, zbuf, bufs, lsems, ssems, zsem):
        s = lax.axis_index("s")
        c = lax.axis_index("c")

        pltpu.sync_copy(dl_hbm, dl_v)
        dl = dl_v[...]
        off_b = jnp.int32(0)
        len_b = jnp.int32(0)
        for i in range(b):
            dli = dl[i]
            off_b = off_b + jnp.where(i < s, dli, 0)
            len_b = len_b + jnp.where(i == s, dli, 0)
        off_b = pl.multiple_of(off_b, 8)

        p0 = c * half
        nvalid = jnp.clip(len_b - p0, 0, half)
        ncopy = nvalid // _CH

        def chunk_src(i):
            return words_hbm.at[pl.ds(off_b + p0 + i * _CH, _CH), :]

        def chunk_dst(i):
            return out_hbm.at[s, pl.ds(p0 + i * _CH, _CH), :]

        @pl.when(ncopy < n_chunks)
        def _():
            pltpu.sync_copy(zpad_hbm, zbuf)

        for i in range(n_chunks):
            @pl.when(i >= ncopy)
            def _(i=i):
                for z in range(_CH // _ZR):
                    r0 = p0 + i * _CH + z * _ZR
                    pltpu.async_copy(
                        zbuf, out_hbm.at[s, pl.ds(r0, _ZR), :], zsem)

        for j in range(min(_NB, n_chunks)):
            @pl.when(j < ncopy)
            def _(j=j):
                pltpu.async_copy(chunk_src(j), bufs[j], lsems[j])

        for i in range(n_chunks):
            k = i % _NB

            @pl.when(i < ncopy)
            def _(i=i, k=k):
                pltpu.make_async_copy(chunk_src(i), bufs[k], lsems[k]).wait()
                pltpu.async_copy(bufs[k], chunk_dst(i), ssems[k])

            j = i + _NB
            if j < n_chunks:
                @pl.when(j < ncopy)
                def _(i=i, j=j, k=k):
                    pltpu.make_async_copy(bufs[k], chunk_dst(i), ssems[k]).wait()
                    pltpu.async_copy(chunk_src(j), bufs[k], lsems[k])

        for i in range(n_chunks):
            k = i % _NB

            @pl.when((i < ncopy) & (i + _NB >= jnp.minimum(ncopy, n_chunks)))
            def _(i=i, k=k):
                pltpu.make_async_copy(bufs[k], chunk_dst(i), ssems[k]).wait()

            @pl.when(i >= ncopy)
            def _(i=i):
                for z in range(_CH // _ZR):
                    r0 = p0 + i * _CH + z * _ZR
                    pltpu.make_async_copy(
                        zbuf, out_hbm.at[s, pl.ds(r0, _ZR), :], zsem).wait()

    return run


def kernel(words_out, doc_lens):
    n, h = words_out.shape
    b = doc_lens.shape[0]
    zpad = jnp.zeros((_ZR, h), jnp.float32)
    run = _build(n, h, b)
    return run(words_out, jnp.asarray(doc_lens, jnp.int32), zpad)
